# baseline (speedup 1.0000x reference)
import jax
import jax.numpy as jnp
from jax import lax
from jax.experimental import pallas as pl
from jax.experimental.pallas import tpu as pltpu
from jax.experimental.pallas import tpu_sc as plsc

_NF = 64
_NS = 100000
_B = 1024
_TEMP = 0.05
_INV_TEMP = 1.0 / _TEMP
_BN = 2000

_NW = 32
_BPW = _B // _NW


def _sc_gather_body(table_hbm, idx_hbm, out_hbm, idx_v, rows_v, sem):
    wid = lax.axis_index("s") * 2 + lax.axis_index("c")
    base = wid * _BPW
    pltpu.sync_copy(idx_hbm.at[pl.ds(base, _BPW)], idx_v)
    pltpu.async_copy(table_hbm.at[idx_v], rows_v, sem).wait()
    pltpu.sync_copy(rows_v, out_hbm.at[pl.ds(base, _BPW)])


def _sc_gather_pairs(features, pair_idx):
    mesh = plsc.VectorSubcoreMesh(core_axis_name="c", subcore_axis_name="s")
    k = pl.kernel(
        _sc_gather_body,
        mesh=mesh,
        out_type=jax.ShapeDtypeStruct((_B, 2 * _NF), jnp.float32),
        scratch_types=[
            pltpu.VMEM((_BPW,), jnp.int32),
            pltpu.VMEM((_BPW, 2 * _NF), jnp.float32),
            pltpu.SemaphoreType.DMA,
        ],
    )
    return k(features.reshape(_NS // 2, 2 * _NF), pair_idx)


def _loss_body(x_ref, tgt_ref, trow_ref, f_ref, out_ref, s_acc):
    i = pl.program_id(0)

    @pl.when(i == 0)
    def _init():
        s_acc[...] = jnp.zeros_like(s_acc)

    x = x_ref[...]
    norm = jnp.sqrt(jnp.sum(x * x, axis=1, keepdims=True))
    xn = x * (_INV_TEMP / jnp.maximum(norm, 1e-12))

    logits = jax.lax.dot_general(
        xn, f_ref[...], (((1,), (1,)), ((), ())))
    s_acc[...] += jnp.sum(jnp.exp(logits), axis=1, keepdims=True)

    @pl.when(i == pl.num_programs(0) - 1)
    def _final():
        trow = trow_ref[...]
        d0 = jnp.sum(xn * trow[:, :_NF], axis=1, keepdims=True)
        d1 = jnp.sum(xn * trow[:, _NF:], axis=1, keepdims=True)
        odd = (tgt_ref[...] & 1) == 1
        tgt = jnp.where(odd, d1, d0)
        lse = jnp.log(s_acc[...])
        out_ref[...] = jnp.mean(lse - tgt).reshape(1, 1)


def kernel(inputs, targets, features):
    tgt_i32 = targets.astype(jnp.int32)
    trows = _sc_gather_pairs(features, tgt_i32 // 2)
    out = pl.pallas_call(
        _loss_body,
        grid=(_NS // _BN,),
        in_specs=[
            pl.BlockSpec((_B, _NF), lambda i: (0, 0)),
            pl.BlockSpec((_B, 1), lambda i: (0, 0)),
            pl.BlockSpec((_B, 2 * _NF), lambda i: (0, 0)),
            pl.BlockSpec((_BN, _NF), lambda i: (i, 0)),
        ],
        out_specs=pl.BlockSpec((1, 1), lambda i: (0, 0)),
        out_shape=jax.ShapeDtypeStruct((1, 1), jnp.float32),
        scratch_shapes=[
            pltpu.VMEM((_B, 1), jnp.float32),
        ],
        compiler_params=pltpu.CompilerParams(
            dimension_semantics=("arbitrary",)),
    )(inputs, tgt_i32.reshape(_B, 1), trows, features)
    return out[0, 0]


# speedup vs baseline: 1.2307x; 1.2307x over previous
import jax
import jax.numpy as jnp
from jax.experimental import pallas as pl
from jax.experimental.pallas import tpu as pltpu

_NF = 64
_NS = 100000
_B = 1024
_TEMP = 0.05
_LOG2E = 1.4426950408889634
_LN2 = 0.6931471805599453
_SCALE = _LOG2E / _TEMP
_BN = 2000


def _loss_body(x_ref, tgt_ref, f_ref, out_ref, s_acc, t_acc):
    i = pl.program_id(0)

    @pl.when(i == 0)
    def _init():
        s_acc[...] = jnp.zeros_like(s_acc)
        t_acc[...] = jnp.zeros_like(t_acc)

    x = x_ref[...]
    norm = jnp.sqrt(jnp.sum(x * x, axis=1, keepdims=True))
    xn = x * (_SCALE / jnp.maximum(norm, 1e-12))

    z = jax.lax.dot_general(
        xn, f_ref[...], (((1,), (1,)), ((), ())))
    s_acc[...] += jnp.sum(jnp.exp2(z), axis=1, keepdims=True)

    cols = i * _BN + jax.lax.broadcasted_iota(jnp.int32, (_B, _BN), 1)
    hit = cols == tgt_ref[...]
    t_acc[...] += jnp.sum(jnp.where(hit, z, 0.0), axis=1, keepdims=True)

    @pl.when(i == pl.num_programs(0) - 1)
    def _final():
        lse2 = jnp.log2(s_acc[...])
        out_ref[...] = (_LN2 * jnp.mean(lse2 - t_acc[...])).reshape(1, 1)


def kernel(inputs, targets, features):
    tgt2 = targets.reshape(_B, 1).astype(jnp.int32)
    out = pl.pallas_call(
        _loss_body,
        grid=(_NS // _BN,),
        in_specs=[
            pl.BlockSpec((_B, _NF), lambda i: (0, 0)),
            pl.BlockSpec((_B, 1), lambda i: (0, 0)),
            pl.BlockSpec((_BN, _NF), lambda i: (i, 0)),
        ],
        out_specs=pl.BlockSpec((1, 1), lambda i: (0, 0)),
        out_shape=jax.ShapeDtypeStruct((1, 1), jnp.float32),
        scratch_shapes=[
            pltpu.VMEM((_B, 1), jnp.float32),
            pltpu.VMEM((_B, 1), jnp.float32),
        ],
        compiler_params=pltpu.CompilerParams(
            dimension_semantics=("arbitrary",)),
    )(inputs, tgt2, features)
    return out[0, 0]


# speedup vs baseline: 1.3070x; 1.0620x over previous
import jax
import jax.numpy as jnp
from jax import lax
from jax.experimental import pallas as pl
from jax.experimental.pallas import tpu as pltpu
from jax.experimental.pallas import tpu_sc as plsc

_NF = 64
_NS = 100000
_B = 1024
_TEMP = 0.05
_LOG2E = 1.4426950408889634
_LN2 = 0.6931471805599453
_SCALE = _LOG2E / _TEMP
_BN = 2000

_NW = 32
_BPW = _B // _NW


def _sc_gather_body(table_hbm, idx_hbm, out_hbm, idx_v, rows_v, sem):
    wid = lax.axis_index("s") * 2 + lax.axis_index("c")
    base = wid * _BPW
    pltpu.sync_copy(idx_hbm.at[pl.ds(base, _BPW)], idx_v)
    lane = lax.iota(jnp.int32, 16)
    copies = []
    for j in range(_BPW):
        grp = idx_v[pl.ds((j // 16) * 16, 16)]
        row = lax.reduce_sum(jnp.where(lane == (j % 16), grp, 0), axes=(0,))
        copies.append(pltpu.async_copy(
            table_hbm.at[pl.ds(row, 1)], rows_v.at[pl.ds(j, 1)], sem))
    for c in copies:
        c.wait()
    pltpu.sync_copy(rows_v, out_hbm.at[pl.ds(base, _BPW)])


def _sc_gather(features, targets):
    mesh = plsc.VectorSubcoreMesh(core_axis_name="c", subcore_axis_name="s")
    k = pl.kernel(
        _sc_gather_body,
        mesh=mesh,
        out_type=jax.ShapeDtypeStruct((_B, _NF), jnp.float32),
        scratch_types=[
            pltpu.VMEM((_BPW,), jnp.int32),
            pltpu.VMEM((_BPW, _NF), jnp.float32),
            pltpu.SemaphoreType.DMA,
        ],
        compiler_params=pltpu.CompilerParams(needs_layout_passes=False),
    )
    return k(features, targets)


def _loss_body(x_ref, trow_ref, f_ref, out_ref, s_acc):
    i = pl.program_id(0)

    @pl.when(i == 0)
    def _init():
        s_acc[...] = jnp.zeros_like(s_acc)

    x = x_ref[...]
    norm = jnp.sqrt(jnp.sum(x * x, axis=1, keepdims=True))
    xn = x * (_SCALE / jnp.maximum(norm, 1e-12))

    z = jax.lax.dot_general(
        xn, f_ref[...], (((1,), (1,)), ((), ())))
    s_acc[...] += jnp.sum(jnp.exp2(z), axis=1, keepdims=True)

    @pl.when(i == pl.num_programs(0) - 1)
    def _final():
        tgt = jnp.sum(xn * trow_ref[...], axis=1, keepdims=True)
        lse2 = jnp.log2(s_acc[...])
        out_ref[...] = (_LN2 * jnp.mean(lse2 - tgt)).reshape(1, 1)


def kernel(inputs, targets, features):
    trows = _sc_gather(features, targets.astype(jnp.int32))
    out = pl.pallas_call(
        _loss_body,
        grid=(_NS // _BN,),
        in_specs=[
            pl.BlockSpec((_B, _NF), lambda i: (0, 0)),
            pl.BlockSpec((_B, _NF), lambda i: (0, 0)),
            pl.BlockSpec((_BN, _NF), lambda i: (i, 0)),
        ],
        out_specs=pl.BlockSpec((1, 1), lambda i: (0, 0)),
        out_shape=jax.ShapeDtypeStruct((1, 1), jnp.float32),
        scratch_shapes=[
            pltpu.VMEM((_B, 1), jnp.float32),
        ],
        compiler_params=pltpu.CompilerParams(
            dimension_semantics=("arbitrary",)),
    )(inputs, trows, features)
    return out[0, 0]


# speedup vs baseline: 1.3326x; 1.0196x over previous
import jax
import jax.numpy as jnp
from jax import lax
from jax.experimental import pallas as pl
from jax.experimental.pallas import tpu as pltpu
from jax.experimental.pallas import tpu_sc as plsc

_NF = 64
_NS = 100000
_B = 1024
_TEMP = 0.05
_LOG2E = 1.4426950408889634
_LN2 = 0.6931471805599453
_SCALE = _LOG2E / _TEMP
_BN = 2000

_NW = 32
_BPW = _B // _NW


def _sc_gather_body(table_hbm, idx_hbm, out_hbm, idx_v, rows_v, sem):
    wid = lax.axis_index("s") * 2 + lax.axis_index("c")
    base = wid * _BPW
    pltpu.sync_copy(idx_hbm.at[pl.ds(base, _BPW)], idx_v)
    lane = lax.iota(jnp.int32, 16)
    copies = []
    for j in range(_BPW):
        grp = idx_v[pl.ds((j // 16) * 16, 16)]
        row = lax.reduce_sum(jnp.where(lane == (j % 16), grp, 0), axes=(0,))
        copies.append(pltpu.async_copy(
            table_hbm.at[pl.ds(row, 1)], rows_v.at[pl.ds(j, 1)], sem))
    for c in copies:
        c.wait()
    pltpu.sync_copy(rows_v, out_hbm.at[pl.ds(base, _BPW)])


def _sc_gather(features, targets):
    mesh = plsc.VectorSubcoreMesh(core_axis_name="c", subcore_axis_name="s")
    k = pl.kernel(
        _sc_gather_body,
        mesh=mesh,
        out_type=jax.ShapeDtypeStruct((_B, _NF), jnp.float32),
        scratch_types=[
            pltpu.VMEM((_BPW,), jnp.int32),
            pltpu.VMEM((_BPW, _NF), jnp.float32),
            pltpu.SemaphoreType.DMA,
        ],
        compiler_params=pltpu.CompilerParams(
            needs_layout_passes=False, use_tc_tiling_on_sc=True),
    )
    return k(features, targets)


def _loss_body(x_ref, trow_ref, f_ref, out_ref, s_acc):
    i = pl.program_id(0)

    @pl.when(i == 0)
    def _init():
        s_acc[...] = jnp.zeros_like(s_acc)

    x = x_ref[...]
    norm = jnp.sqrt(jnp.sum(x * x, axis=1, keepdims=True))
    xn = x * (_SCALE / jnp.maximum(norm, 1e-12))

    z = jax.lax.dot_general(
        xn, f_ref[...], (((1,), (1,)), ((), ())))
    s_acc[...] += jnp.sum(jnp.exp2(z), axis=1, keepdims=True)

    @pl.when(i == pl.num_programs(0) - 1)
    def _final():
        tgt = jnp.sum(xn * trow_ref[...], axis=1, keepdims=True)
        lse2 = jnp.log2(s_acc[...])
        out_ref[...] = (_LN2 * jnp.mean(lse2 - tgt)).reshape(1, 1)


def kernel(inputs, targets, features):
    trows = _sc_gather(features, targets.astype(jnp.int32))
    out = pl.pallas_call(
        _loss_body,
        grid=(_NS // _BN,),
        in_specs=[
            pl.BlockSpec((_B, _NF), lambda i: (0, 0)),
            pl.BlockSpec((_B, _NF), lambda i: (0, 0)),
            pl.BlockSpec((_BN, _NF), lambda i: (i, 0)),
        ],
        out_specs=pl.BlockSpec((1, 1), lambda i: (0, 0)),
        out_shape=jax.ShapeDtypeStruct((1, 1), jnp.float32),
        scratch_shapes=[
            pltpu.VMEM((_B, 1), jnp.float32),
        ],
        compiler_params=pltpu.CompilerParams(
            dimension_semantics=("arbitrary",)),
    )(inputs, trows, features)
    return out[0, 0]
